# Initial kernel scaffold; baseline (speedup 1.0000x reference)
import jax
import jax.numpy as jnp
from jax.experimental import pallas as pl


def kernel(x, edge_index, edge_attr, h, batch, W, b, ln_scale, ln_bias):
    raise NotImplementedError("write your pallas kernel here")


# speedup vs baseline: 30.5605x; 30.5605x over previous
import functools

import jax
import jax.numpy as jnp
from jax import lax
from jax.experimental import pallas as pl
from jax.experimental.pallas import tpu as pltpu
from jax.experimental.pallas import tpu_sc as plsc

N = 10000
D = 128
LN_EPS = 1e-5

NC = 2
NS = 16
NW = NC * NS
CHUNK = 128
CPT = 80
EPT = CHUNK * CPT
EPAD = EPT * NW
NP = 10240
RPT = NP // NS
NBLK = 8
KB = CPT // NBLK

_mesh = plsc.VectorSubcoreMesh(core_axis_name="c", subcore_axis_name="s")


@functools.partial(
    pl.kernel,
    out_type=jax.ShapeDtypeStruct((NC * NP,), jnp.float32),
    mesh=_mesh,
    scratch_types=[
        pltpu.VMEM((NBLK, KB, CHUNK), jnp.int32),
        pltpu.VMEM((CHUNK,), jnp.float32),
        pltpu.VMEM((RPT,), jnp.float32),
        pltpu.VMEM_SHARED((NP,), jnp.float32),
        pltpu.SemaphoreType.DMA,
    ],
)
def _deg_kernel(dst_hbm, deg_out, dst_v, ones_v, zer_v, acc_s, sem):
    c = lax.axis_index("c")
    s = lax.axis_index("s")
    wid = c * NS + s
    cp = pltpu.async_copy(dst_hbm.at[wid], dst_v, sem)
    for k in range(CHUNK // 16):
        ones_v[pl.ds(k * 16, 16)] = jnp.ones((16,), jnp.float32)

    def zbody(k, _):
        zer_v[pl.ds(k * 16, 16)] = jnp.zeros((16,), jnp.float32)
        return 0

    lax.fori_loop(0, RPT // 16, zbody, 0)
    pltpu.sync_copy(zer_v, acc_s.at[pl.ds(s * RPT, RPT)])
    cp.wait()
    plsc.subcore_barrier()

    def body(j, _):
        pltpu.sync_copy(ones_v, acc_s.at[dst_v.at[j // KB, j % KB]], add=True)
        return 0

    lax.fori_loop(0, CPT, body, 0)
    plsc.subcore_barrier()
    pltpu.sync_copy(acc_s.at[pl.ds(s * RPT, RPT)],
                    deg_out.at[pl.ds(c * NP + s * RPT, RPT)])


@functools.partial(
    pl.kernel,
    out_type=jax.ShapeDtypeStruct((NC, NP, D), jnp.float32),
    mesh=_mesh,
    scratch_types=[
        pltpu.VMEM((2, KB, CHUNK), jnp.int32),
        pltpu.VMEM((2, KB, CHUNK), jnp.int32),
        pltpu.VMEM((CHUNK, D), jnp.float32),
        pltpu.VMEM((CHUNK, D), jnp.float32),
        pltpu.VMEM_SHARED((NP, D), jnp.float32),
        pltpu.SemaphoreType.DMA,
        pltpu.SemaphoreType.DMA,
        pltpu.SemaphoreType.DMA,
    ],
)
def _conv_kernel(y_hbm, src_hbm, dst_hbm, out_hbm,
                 src_v, dst_v, buf0, buf1, acc_s, sem0, sem1, semi):
    c = lax.axis_index("c")
    s = lax.axis_index("s")
    wid = c * NS + s
    cps = pltpu.async_copy(src_hbm.at[wid, 0], src_v.at[0], semi)
    cpd = pltpu.async_copy(dst_hbm.at[wid, 0], dst_v.at[0], semi)

    def zbody(k, _):
        buf0[k // (D // 16), pl.ds((k % (D // 16)) * 16, 16)] = (
            jnp.zeros((16,), jnp.float32))
        return 0

    lax.fori_loop(0, CHUNK * D // 16, zbody, 0)
    for t in range(RPT // CHUNK):
        pltpu.sync_copy(buf0, acc_s.at[pl.ds(s * RPT + t * CHUNK, CHUNK)])
    cps.wait()
    cpd.wait()
    plsc.subcore_barrier()

    for ib in range(NBLK):
        cur = ib % 2
        if ib + 1 < NBLK:
            pltpu.async_copy(src_hbm.at[wid, ib + 1], src_v.at[1 - cur], semi)
            pltpu.async_copy(dst_hbm.at[wid, ib + 1], dst_v.at[1 - cur], semi)

        def body(jj, _, cur=cur):
            j0 = 2 * jj
            c0 = pltpu.async_copy(y_hbm.at[src_v.at[cur, j0]], buf0, sem0)
            c1 = pltpu.async_copy(y_hbm.at[src_v.at[cur, j0 + 1]], buf1, sem1)
            c0.wait()
            pltpu.sync_copy(buf0, acc_s.at[dst_v.at[cur, j0]], add=True)
            c1.wait()
            pltpu.sync_copy(buf1, acc_s.at[dst_v.at[cur, j0 + 1]], add=True)
            return 0

        lax.fori_loop(0, KB // 2, body, 0)
        if ib + 1 < NBLK:
            pltpu.make_async_copy(src_hbm.at[wid, ib + 1],
                                  src_v.at[1 - cur], semi).wait()
            pltpu.make_async_copy(dst_hbm.at[wid, ib + 1],
                                  dst_v.at[1 - cur], semi).wait()
    plsc.subcore_barrier()
    pltpu.sync_copy(acc_s.at[pl.ds(s * RPT, RPT)],
                    out_hbm.at[c, pl.ds(s * RPT, RPT)])


def _ln_mm_body(x_ref, w_ref, sc_ref, bi_ref, xw_ref):
    xv = x_ref[...]
    mu = jnp.mean(xv, axis=1, keepdims=True)
    xc = xv - mu
    var = jnp.mean(xc * xc, axis=1, keepdims=True)
    xn = xc * lax.rsqrt(var + LN_EPS) * sc_ref[...] + bi_ref[...]
    xw_ref[...] = jnp.dot(xn, w_ref[...], preferred_element_type=jnp.float32)


def _scale_body(xw_ref, x_ref, deg_ref, b_ref, y_ref, r_ref):
    dinv = lax.rsqrt(deg_ref[...])
    xw = xw_ref[...]
    y_ref[...] = xw * dinv
    r_ref[...] = x_ref[...] + b_ref[...] + dinv * dinv * xw


def _combine_body(p_ref, deg_ref, r_ref, o_ref):
    dinv = lax.rsqrt(deg_ref[...])
    agg = dinv * (p_ref[0] + p_ref[1]) + r_ref[...]
    o_ref[...] = jnp.maximum(agg, 0.0)


_BLK = 1000
_GRID = N // _BLK


def kernel(x, edge_index, edge_attr, h, batch, W, b, ln_scale, ln_bias):
    del edge_attr, batch
    f32 = jnp.float32

    e = edge_index.shape[1]
    pad = EPAD - e
    ar = jnp.arange(pad, dtype=edge_index.dtype)
    src = jnp.concatenate([edge_index[0], (ar * 37) % N]).reshape(NW, NBLK, KB, CHUNK)
    dst = jnp.concatenate([edge_index[1], N + ar % (NP - N)]).reshape(NW, NBLK, KB, CHUNK)

    xw = pl.pallas_call(
        _ln_mm_body,
        grid=(_GRID,),
        in_specs=[
            pl.BlockSpec((_BLK, D), lambda j: (j, 0)),
            pl.BlockSpec((D, D), lambda j: (0, 0)),
            pl.BlockSpec((1, D), lambda j: (0, 0)),
            pl.BlockSpec((1, D), lambda j: (0, 0)),
        ],
        out_specs=pl.BlockSpec((_BLK, D), lambda j: (j, 0)),
        out_shape=jax.ShapeDtypeStruct((N, D), f32),
    )(x, W, ln_scale.reshape(1, D), ln_bias.reshape(1, D))

    deg_parts = _deg_kernel(dst)
    dp = deg_parts.reshape(NC, NP)
    deg_col = (dp[0, :N] + dp[1, :N] + 1.0).reshape(N, 1)

    y, r = pl.pallas_call(
        _scale_body,
        grid=(_GRID,),
        in_specs=[
            pl.BlockSpec((_BLK, D), lambda j: (j, 0)),
            pl.BlockSpec((_BLK, D), lambda j: (j, 0)),
            pl.BlockSpec((_BLK, 1), lambda j: (j, 0)),
            pl.BlockSpec((1, D), lambda j: (0, 0)),
        ],
        out_specs=[
            pl.BlockSpec((_BLK, D), lambda j: (j, 0)),
            pl.BlockSpec((_BLK, D), lambda j: (j, 0)),
        ],
        out_shape=[
            jax.ShapeDtypeStruct((N, D), f32),
            jax.ShapeDtypeStruct((N, D), f32),
        ],
    )(xw, x, deg_col, b.reshape(1, D))

    parts = _conv_kernel(y, src, dst)

    out = pl.pallas_call(
        _combine_body,
        grid=(_GRID,),
        in_specs=[
            pl.BlockSpec((NC, _BLK, D), lambda j: (0, j, 0)),
            pl.BlockSpec((_BLK, 1), lambda j: (j, 0)),
            pl.BlockSpec((_BLK, D), lambda j: (j, 0)),
        ],
        out_specs=pl.BlockSpec((_BLK, D), lambda j: (j, 0)),
        out_shape=jax.ShapeDtypeStruct((N, D), f32),
    )(parts, deg_col, r)

    return (out, h)


# speedup vs baseline: 31.3707x; 1.0265x over previous
import functools

import jax
import jax.numpy as jnp
from jax import lax
from jax.experimental import pallas as pl
from jax.experimental.pallas import tpu as pltpu
from jax.experimental.pallas import tpu_sc as plsc

N = 10000
D = 128
LN_EPS = 1e-5

NC = 2
NS = 16
NW = NC * NS
CHUNK = 128
CPT = 80
EPT = CHUNK * CPT
EPAD = EPT * NW
NP = 10240
RPT = NP // NS
NBLK = 8
KB = CPT // NBLK

_mesh = plsc.VectorSubcoreMesh(core_axis_name="c", subcore_axis_name="s")


@functools.partial(
    pl.kernel,
    out_type=jax.ShapeDtypeStruct((NC * NP,), jnp.float32),
    mesh=_mesh,
    scratch_types=[
        pltpu.VMEM((NBLK, KB, CHUNK), jnp.int32),
        pltpu.VMEM((CHUNK,), jnp.float32),
        pltpu.VMEM((RPT,), jnp.float32),
        pltpu.VMEM_SHARED((NP,), jnp.float32),
        pltpu.SemaphoreType.DMA,
    ],
)
def _deg_kernel(dst_hbm, deg_out, dst_v, ones_v, zer_v, acc_s, sem):
    c = lax.axis_index("c")
    s = lax.axis_index("s")
    wid = c * NS + s
    cp = pltpu.async_copy(dst_hbm.at[wid], dst_v, sem)
    for k in range(CHUNK // 16):
        ones_v[pl.ds(k * 16, 16)] = jnp.ones((16,), jnp.float32)

    def zbody(k, _):
        zer_v[pl.ds(k * 16, 16)] = jnp.zeros((16,), jnp.float32)
        return 0

    lax.fori_loop(0, RPT // 16, zbody, 0)
    pltpu.sync_copy(zer_v, acc_s.at[pl.ds(s * RPT, RPT)])
    cp.wait()
    plsc.subcore_barrier()

    def body(j, _):
        pltpu.sync_copy(ones_v, acc_s.at[dst_v.at[j // KB, j % KB]], add=True)
        return 0

    lax.fori_loop(0, CPT, body, 0)
    plsc.subcore_barrier()
    pltpu.sync_copy(acc_s.at[pl.ds(s * RPT, RPT)],
                    deg_out.at[pl.ds(c * NP + s * RPT, RPT)])


HKB = KB // 2


@functools.partial(
    pl.kernel,
    out_type=jax.ShapeDtypeStruct((NC, NP, D), jnp.float32),
    mesh=_mesh,
    scratch_types=[
        pltpu.VMEM((CPT, CHUNK), jnp.int32),
        pltpu.VMEM((2, KB, CHUNK), jnp.int32),
        pltpu.VMEM((CHUNK, D), jnp.float32),
        pltpu.VMEM((CHUNK, D), jnp.float32),
        pltpu.VMEM_SHARED((NP, D), jnp.float32),
        pltpu.SemaphoreType.DMA,
        pltpu.SemaphoreType.DMA,
        pltpu.SemaphoreType.DMA,
        pltpu.SemaphoreType.DMA,
        pltpu.SemaphoreType.DMA,
    ],
)
def _conv_kernel(y_hbm, src_hbm, dst_hbm, out_hbm, src_v, dst_v, buf0, buf1,
                 acc_s, semg0, semg1, sems0, sems1, semi):
    c = lax.axis_index("c")
    s = lax.axis_index("s")
    wid = c * NS + s
    cps = pltpu.async_copy(src_hbm.at[wid], src_v, semi)
    cpd = pltpu.async_copy(dst_hbm.at[wid, 0], dst_v.at[0], semi)

    def zbody(k, _):
        buf0[k // (D // 16), pl.ds((k % (D // 16)) * 16, 16)] = (
            jnp.zeros((16,), jnp.float32))
        return 0

    lax.fori_loop(0, CHUNK * D // 16, zbody, 0)
    for t in range(RPT // CHUNK):
        pltpu.sync_copy(buf0, acc_s.at[pl.ds(s * RPT + t * CHUNK, CHUNK)])
    cps.wait()
    cpd.wait()
    plsc.subcore_barrier()
    pltpu.async_copy(y_hbm.at[src_v.at[0]], buf0, semg0)
    pltpu.async_copy(y_hbm.at[src_v.at[1]], buf1, semg1)

    def body(p, _):
        a = 2 * p
        blk = p // HKB
        cur = blk % 2
        a_loc = (p % HKB) * 2
        at_blk_start = p % HKB == 0

        @pl.when(jnp.logical_and(at_blk_start, blk + 1 < NBLK))
        def _():
            pltpu.async_copy(dst_hbm.at[wid, blk + 1], dst_v.at[1 - cur], semi)

        @pl.when(jnp.logical_and(at_blk_start, blk > 0))
        def _():
            pltpu.make_async_copy(dst_hbm.at[wid, blk],
                                  dst_v.at[cur], semi).wait()

        pltpu.make_async_copy(y_hbm.at[src_v.at[a]], buf0, semg0).wait()
        s0 = pltpu.async_copy(buf0, acc_s.at[dst_v.at[cur, a_loc]], sems0,
                              add=True)
        pltpu.make_async_copy(y_hbm.at[src_v.at[a + 1]], buf1, semg1).wait()
        s1 = pltpu.async_copy(buf1, acc_s.at[dst_v.at[cur, a_loc + 1]], sems1,
                              add=True)
        s0.wait()

        @pl.when(a + 2 < CPT)
        def _():
            pltpu.async_copy(y_hbm.at[src_v.at[a + 2]], buf0, semg0)

        s1.wait()

        @pl.when(a + 3 < CPT)
        def _():
            pltpu.async_copy(y_hbm.at[src_v.at[a + 3]], buf1, semg1)

        return 0

    lax.fori_loop(0, CPT // 2, body, 0)
    plsc.subcore_barrier()
    pltpu.sync_copy(acc_s.at[pl.ds(s * RPT, RPT)],
                    out_hbm.at[c, pl.ds(s * RPT, RPT)])


def _ln_mm_scale_body(x_ref, w_ref, sc_ref, bi_ref, deg_ref, b_ref,
                      y_ref, r_ref):
    xv = x_ref[...]
    mu = jnp.mean(xv, axis=1, keepdims=True)
    xc = xv - mu
    var = jnp.mean(xc * xc, axis=1, keepdims=True)
    xn = xc * lax.rsqrt(var + LN_EPS) * sc_ref[...] + bi_ref[...]
    xw = jnp.dot(xn, w_ref[...], preferred_element_type=jnp.float32)
    dinv = lax.rsqrt(deg_ref[...])
    y_ref[...] = xw * dinv
    r_ref[...] = xv + b_ref[...] + dinv * dinv * xw


def _combine_body(p_ref, deg_ref, r_ref, o_ref):
    dinv = lax.rsqrt(deg_ref[...])
    agg = dinv * (p_ref[0] + p_ref[1]) + r_ref[...]
    o_ref[...] = jnp.maximum(agg, 0.0)


_BLK = 1000
_GRID = N // _BLK


def kernel(x, edge_index, edge_attr, h, batch, W, b, ln_scale, ln_bias):
    del edge_attr, batch
    f32 = jnp.float32

    e = edge_index.shape[1]
    pad = EPAD - e
    ar = jnp.arange(pad, dtype=edge_index.dtype)
    src = jnp.concatenate([edge_index[0], (ar * 37) % N]).reshape(NW, CPT, CHUNK)
    dst = jnp.concatenate([edge_index[1], N + ar % (NP - N)]).reshape(NW, NBLK, KB, CHUNK)

    deg_parts = _deg_kernel(dst)
    dp = deg_parts.reshape(NC, NP)
    deg_col = (dp[0, :N] + dp[1, :N] + 1.0).reshape(N, 1)

    y, r = pl.pallas_call(
        _ln_mm_scale_body,
        grid=(_GRID,),
        in_specs=[
            pl.BlockSpec((_BLK, D), lambda j: (j, 0)),
            pl.BlockSpec((D, D), lambda j: (0, 0)),
            pl.BlockSpec((1, D), lambda j: (0, 0)),
            pl.BlockSpec((1, D), lambda j: (0, 0)),
            pl.BlockSpec((_BLK, 1), lambda j: (j, 0)),
            pl.BlockSpec((1, D), lambda j: (0, 0)),
        ],
        out_specs=[
            pl.BlockSpec((_BLK, D), lambda j: (j, 0)),
            pl.BlockSpec((_BLK, D), lambda j: (j, 0)),
        ],
        out_shape=[
            jax.ShapeDtypeStruct((N, D), f32),
            jax.ShapeDtypeStruct((N, D), f32),
        ],
    )(x, W, ln_scale.reshape(1, D), ln_bias.reshape(1, D), deg_col,
      b.reshape(1, D))

    parts = _conv_kernel(y, src, dst)

    out = pl.pallas_call(
        _combine_body,
        grid=(_GRID,),
        in_specs=[
            pl.BlockSpec((NC, _BLK, D), lambda j: (0, j, 0)),
            pl.BlockSpec((_BLK, 1), lambda j: (j, 0)),
            pl.BlockSpec((_BLK, D), lambda j: (j, 0)),
        ],
        out_specs=pl.BlockSpec((_BLK, D), lambda j: (j, 0)),
        out_shape=jax.ShapeDtypeStruct((N, D), f32),
    )(parts, deg_col, r)

    return (out, h)
